# Initial kernel scaffold; baseline (speedup 1.0000x reference)
import jax
import jax.numpy as jnp
from jax.experimental import pallas as pl


def kernel(x, edge_index, params):
    raise NotImplementedError("write your pallas kernel here")


# speedup vs baseline: 1.1514x; 1.1514x over previous
import functools

import jax
import jax.numpy as jnp
from jax import lax
from jax.experimental import pallas as pl
from jax.experimental.pallas import tpu as pltpu
from jax.experimental.pallas import tpu_sc as plsc

N = 10000
E = 160000
D = 256
H = 128
NS = 16
NPB = 625
P = 12000
K = 80
NCHUNK = P // K
TRASH = N
AROWS = N + 16
RPT = 624


def _sc_scatter_body(x_hbm, src2_hbm, dst_hbm, zeros_hbm, out_hbm,
                     src_v, dst_v, rows_v, agg_sh, sem):
    c = lax.axis_index("c")
    s = lax.axis_index("s")

    pltpu.sync_copy(zeros_hbm.at[pl.ds(s * RPT, RPT)],
                    agg_sh.at[pl.ds(s * RPT, RPT)])

    @pl.when(s < 2)
    def _():
        pltpu.sync_copy(zeros_hbm.at[pl.ds(NS * RPT + s * 8, 8)],
                        agg_sh.at[pl.ds(NS * RPT + s * 8, 8)])

    plsc.subcore_barrier()

    ebase = s * P

    @pl.loop(0, NCHUNK)
    def _(j):
        base = ebase + j * K
        pltpu.sync_copy(src2_hbm.at[pl.ds(c * (NS * P) + base, K)], src_v)
        pltpu.sync_copy(dst_hbm.at[pl.ds(base, K)], dst_v)
        pltpu.async_copy(x_hbm.at[src_v], rows_v, sem).wait()
        pltpu.sync_copy(rows_v, agg_sh.at[dst_v], add=True)

    plsc.subcore_barrier()
    pltpu.sync_copy(agg_sh.at[pl.ds(s * RPT, RPT)],
                    out_hbm.at[pl.ds(c * N + s * RPT, RPT)])

    @pl.when(s < 2)
    def _():
        pltpu.sync_copy(agg_sh.at[pl.ds(NS * RPT + s * 8, 8)],
                        out_hbm.at[pl.ds(c * N + NS * RPT + s * 8, 8)])


@jax.jit
def _sc_scatter(x_flat, src2, dst, zeros):
    mesh = plsc.VectorSubcoreMesh(core_axis_name="c", subcore_axis_name="s",
                                  num_cores=2, num_subcores=NS)
    fn = pl.kernel(
        _sc_scatter_body,
        out_type=jax.ShapeDtypeStruct((2 * N, H), jnp.float32),
        mesh=mesh,
        scratch_types=[
            pltpu.VMEM((K,), jnp.int32),
            pltpu.VMEM((K,), jnp.int32),
            pltpu.VMEM((K, H), jnp.float32),
            pltpu.VMEM_SHARED((AROWS, H), jnp.float32),
            pltpu.SemaphoreType.DMA,
        ],
    )
    return fn(x_flat, src2, dst, zeros)


RB = 2000
NB = N // RB


def _dot_mx(a, b):
    return jnp.dot(a, b, preferred_element_type=jnp.float32)


def _leaky(h):
    return jnp.where(h >= 0.0, h, 0.01 * h)


def _col_sum(h_ref, f):
    def half(lo):
        def body(i, acc):
            return acc + f(h_ref[pl.ds(lo + i * 8, 8), :])
        acc = lax.fori_loop(0, (N // 2) // 8, body,
                            jnp.zeros((8, D), jnp.float32))
        t = acc[0:4, :] + acc[4:8, :]
        t = t[0:2, :] + t[2:4, :]
        return t[0:1, :] + t[1:2, :]
    return half(0) + half(N // 2)


def _bn_stats(h_ref):
    mu = _col_sum(h_ref, lambda t: t) * (1.0 / N)
    var = _col_sum(h_ref, lambda t: (t - mu) * (t - mu)) * (1.0 / N)
    return mu, var


def _bn_apply(h, mu, var, g, b):
    return (h - mu) / jnp.sqrt(var + 1e-5) * g + b


def _mm1_body(x_ref, a_ref, w_ref, b_ref, eps_ref, o_ref):
    e = 1.0 + eps_ref[0, 0]
    for i in range(NB):
        r = pl.ds(i * RB, RB)
        rt = pl.ds(N + i * RB, RB)
        u = jnp.concatenate([e * x_ref[r, :] + a_ref[r, :],
                             e * x_ref[rt, :] + a_ref[rt, :]], axis=1)
        o_ref[r, :] = _dot_mx(u, w_ref[...]) + b_ref[...]


@jax.jit
def _tc_mm1(x_flat, agg_flat, w1, b1, eps):
    return pl.pallas_call(
        _mm1_body,
        out_shape=jax.ShapeDtypeStruct((N, D), jnp.float32),
    )(x_flat, agg_flat, w1, b1.reshape(1, D), eps.reshape(1, 1))


def _bn_mm2_body(h_ref, g1_ref, be1_ref, w2_ref, b2_ref, g2_ref, be2_ref,
                 o_ref, y_ref, *, do_bn2):
    mu, var = _bn_stats(h_ref)
    h = _leaky(_bn_apply(h_ref[...], mu, var, g1_ref[...], be1_ref[...]))
    for i in range(NB):
        r = pl.ds(i * RB, RB)
        y_ref[r, :] = _dot_mx(h[i * RB:(i + 1) * RB, :], w2_ref[...]) + b2_ref[...]
    y = y_ref[...]
    if do_bn2:
        mu2, var2 = _bn_stats(y_ref)
        y = _bn_apply(y, mu2, var2, g2_ref[...], be2_ref[...])
    y = _leaky(y)
    o_ref[:N, :] = y[:, :H]
    o_ref[N:, :] = y[:, H:]


@functools.partial(jax.jit, static_argnames=("do_bn2",))
def _tc_bn_mm2(h1, g1, be1, w2, b2, g2, be2, do_bn2):
    return pl.pallas_call(
        functools.partial(_bn_mm2_body, do_bn2=do_bn2),
        out_shape=jax.ShapeDtypeStruct((2 * N, H), jnp.float32),
        scratch_shapes=[pltpu.VMEM((N, D), jnp.float32)],
    )(h1, g1.reshape(1, D), be1.reshape(1, D), w2, b2.reshape(1, D),
      g2.reshape(1, D), be2.reshape(1, D))


def _cls_body(x_ref, w1_ref, b1_ref, g1_ref, be1_ref, w2_ref, b2_ref,
              g2_ref, be2_ref, w3_ref, b3_ref, o_ref, t_ref):
    for i in range(NB):
        r = pl.ds(i * RB, RB)
        rt = pl.ds(N + i * RB, RB)
        u = jnp.concatenate([x_ref[r, :], x_ref[rt, :]], axis=1)
        t_ref[r, :] = _dot_mx(u, w1_ref[...]) + b1_ref[...]
    mu, var = _bn_stats(t_ref)
    h = _leaky(_bn_apply(t_ref[...], mu, var, g1_ref[...], be1_ref[...]))
    for i in range(NB):
        r = pl.ds(i * RB, RB)
        t_ref[r, :] = _dot_mx(h[i * RB:(i + 1) * RB, :], w2_ref[...]) + b2_ref[...]
    mu2, var2 = _bn_stats(t_ref)
    h2 = _leaky(_bn_apply(t_ref[...], mu2, var2, g2_ref[...], be2_ref[...]))
    h2b = h2.astype(jnp.bfloat16).astype(jnp.float32)
    w3b = w3_ref[...].astype(jnp.bfloat16).astype(jnp.float32)
    o_ref[...] = jnp.sum(h2b * w3b, axis=1, keepdims=True) + b3_ref[0, 0]


@jax.jit
def _tc_cls(x_flat, w1, b1, g1, be1, w2, b2, g2, be2, w3, b3):
    return pl.pallas_call(
        _cls_body,
        out_shape=jax.ShapeDtypeStruct((N, 1), jnp.float32),
        scratch_shapes=[pltpu.VMEM((N, D), jnp.float32)],
    )(x_flat, w1, b1.reshape(1, D), g1.reshape(1, D), be1.reshape(1, D),
      w2, b2.reshape(1, D), g2.reshape(1, D), be2.reshape(1, D),
      w3.reshape(1, D), b3.reshape(1, 1))


def kernel(x, edge_index, params):
    src = edge_index[0].astype(jnp.int32)
    dst = edge_index[1].astype(jnp.int32)

    bucket = dst // NPB
    perm = jnp.argsort(bucket, stable=True)
    srcp = src[perm]
    dstp = dst[perm]
    bsort = bucket[perm]
    eidx = jnp.arange(E, dtype=jnp.int32)
    start = jnp.searchsorted(bsort, jnp.arange(NS, dtype=jnp.int32)).astype(jnp.int32)
    rank = eidx - start[bsort]
    pos = jnp.where(rank < P, bsort * P + rank, NS * P)
    fill_src = (jnp.arange(NS * P + 8, dtype=jnp.int32) * 7919) % N
    src_pad = fill_src.at[pos].set(srcp)[:NS * P]
    dst_pad = jnp.full((NS * P + 8,), TRASH, jnp.int32).at[pos].set(dstp)[:NS * P]
    src2 = jnp.concatenate([src_pad, src_pad + N])

    x_flat = jnp.concatenate([x[:, :H], x[:, H:]], axis=0)
    zeros = jnp.zeros((N, H), jnp.float32)

    for i in range(3):
        agg_flat = _sc_scatter(x_flat, src2, dst_pad, zeros)
        h1 = _tc_mm1(x_flat, agg_flat, params[f"conv{i}_W1"],
                     params[f"conv{i}_b1"], params[f"conv{i}_eps"])
        x_flat = _tc_bn_mm2(h1, params[f"conv{i}_g1"], params[f"conv{i}_be1"],
                            params[f"conv{i}_W2"], params[f"conv{i}_b2"],
                            params.get(f"bn{i}_g", params[f"conv{i}_g1"]),
                            params.get(f"bn{i}_b", params[f"conv{i}_be1"]),
                            do_bn2=(i != 2))

    out = _tc_cls(x_flat, params["cls_W1"], params["cls_b1"],
                  params["cls_g1"], params["cls_be1"],
                  params["cls_W2"], params["cls_b2"],
                  params["cls_g2"], params["cls_be2"],
                  params["cls_W3"], params["cls_b3"])
    return out.reshape(-1)


# speedup vs baseline: 1.3101x; 1.1379x over previous
import functools

import jax
import jax.numpy as jnp
from jax import lax
from jax.experimental import pallas as pl
from jax.experimental.pallas import tpu as pltpu
from jax.experimental.pallas import tpu_sc as plsc

N = 10000
E = 160000
D = 256
H = 128
NS = 16
NPB = 625
P = 10800
K = 80
NCHUNK = P // K
TRASH = N
AROWS = N + 16
RPT = 624


def _sc_scatter_body(x_hbm, src2_hbm, dst_hbm, zeros_hbm, out_hbm,
                     src_v, dst_v, rows_v, agg_sh, sem):
    c = lax.axis_index("c")
    s = lax.axis_index("s")

    pltpu.sync_copy(zeros_hbm.at[pl.ds(s * RPT, RPT)],
                    agg_sh.at[pl.ds(s * RPT, RPT)])

    @pl.when(s < 2)
    def _():
        pltpu.sync_copy(zeros_hbm.at[pl.ds(NS * RPT + s * 8, 8)],
                        agg_sh.at[pl.ds(NS * RPT + s * 8, 8)])

    plsc.subcore_barrier()

    ebase = s * P

    @pl.loop(0, NCHUNK)
    def _(j):
        base = ebase + j * K
        pltpu.sync_copy(src2_hbm.at[pl.ds(c * (NS * P) + base, K)], src_v)
        pltpu.sync_copy(dst_hbm.at[pl.ds(base, K)], dst_v)
        pltpu.async_copy(x_hbm.at[src_v], rows_v, sem).wait()
        pltpu.sync_copy(rows_v, agg_sh.at[dst_v], add=True)

    plsc.subcore_barrier()
    pltpu.sync_copy(agg_sh.at[pl.ds(s * RPT, RPT)],
                    out_hbm.at[pl.ds(c * N + s * RPT, RPT)])

    @pl.when(s < 2)
    def _():
        pltpu.sync_copy(agg_sh.at[pl.ds(NS * RPT + s * 8, 8)],
                        out_hbm.at[pl.ds(c * N + NS * RPT + s * 8, 8)])


@jax.jit
def _sc_scatter(x_flat, src2, dst, zeros):
    mesh = plsc.VectorSubcoreMesh(core_axis_name="c", subcore_axis_name="s",
                                  num_cores=2, num_subcores=NS)
    fn = pl.kernel(
        _sc_scatter_body,
        out_type=jax.ShapeDtypeStruct((2 * N, H), jnp.float32),
        mesh=mesh,
        scratch_types=[
            pltpu.VMEM((K,), jnp.int32),
            pltpu.VMEM((K,), jnp.int32),
            pltpu.VMEM((K, H), jnp.float32),
            pltpu.VMEM_SHARED((AROWS, H), jnp.float32),
            pltpu.SemaphoreType.DMA,
        ],
    )
    return fn(x_flat, src2, dst, zeros)


RB = 2000
NB = N // RB


def _dot_mx(a, b):
    return jnp.dot(a, b, preferred_element_type=jnp.float32)


def _leaky(h):
    return jnp.where(h >= 0.0, h, 0.01 * h)


def _col_sum(h_ref, f):
    def half(lo):
        def body(i, acc):
            return acc + f(h_ref[pl.ds(lo + i * 8, 8), :])
        acc = lax.fori_loop(0, (N // 2) // 8, body,
                            jnp.zeros((8, D), jnp.float32))
        t = acc[0:4, :] + acc[4:8, :]
        t = t[0:2, :] + t[2:4, :]
        return t[0:1, :] + t[1:2, :]
    return half(0) + half(N // 2)


def _bn_stats(h_ref):
    mu = _col_sum(h_ref, lambda t: t) * (1.0 / N)
    var = _col_sum(h_ref, lambda t: (t - mu) * (t - mu)) * (1.0 / N)
    return mu, var


def _bn_apply(h, mu, var, g, b):
    return (h - mu) / jnp.sqrt(var + 1e-5) * g + b


def _mm1_body(x_ref, a_ref, w_ref, b_ref, eps_ref, o_ref):
    e = 1.0 + eps_ref[0, 0]
    for i in range(NB):
        r = pl.ds(i * RB, RB)
        rt = pl.ds(N + i * RB, RB)
        u = jnp.concatenate([e * x_ref[r, :] + a_ref[r, :],
                             e * x_ref[rt, :] + a_ref[rt, :]], axis=1)
        o_ref[r, :] = _dot_mx(u, w_ref[...]) + b_ref[...]


@jax.jit
def _tc_mm1(x_flat, agg_flat, w1, b1, eps):
    return pl.pallas_call(
        _mm1_body,
        out_shape=jax.ShapeDtypeStruct((N, D), jnp.float32),
    )(x_flat, agg_flat, w1, b1.reshape(1, D), eps.reshape(1, 1))


def _bn_mm2_body(h_ref, g1_ref, be1_ref, w2_ref, b2_ref, g2_ref, be2_ref,
                 o_ref, y_ref, *, do_bn2):
    mu, var = _bn_stats(h_ref)
    h = _leaky(_bn_apply(h_ref[...], mu, var, g1_ref[...], be1_ref[...]))
    for i in range(NB):
        r = pl.ds(i * RB, RB)
        y_ref[r, :] = _dot_mx(h[i * RB:(i + 1) * RB, :], w2_ref[...]) + b2_ref[...]
    y = y_ref[...]
    if do_bn2:
        mu2, var2 = _bn_stats(y_ref)
        y = _bn_apply(y, mu2, var2, g2_ref[...], be2_ref[...])
    y = _leaky(y)
    o_ref[:N, :] = y[:, :H]
    o_ref[N:, :] = y[:, H:]


@functools.partial(jax.jit, static_argnames=("do_bn2",))
def _tc_bn_mm2(h1, g1, be1, w2, b2, g2, be2, do_bn2):
    return pl.pallas_call(
        functools.partial(_bn_mm2_body, do_bn2=do_bn2),
        out_shape=jax.ShapeDtypeStruct((2 * N, H), jnp.float32),
        scratch_shapes=[pltpu.VMEM((N, D), jnp.float32)],
    )(h1, g1.reshape(1, D), be1.reshape(1, D), w2, b2.reshape(1, D),
      g2.reshape(1, D), be2.reshape(1, D))


def _cls_body(x_ref, w1_ref, b1_ref, g1_ref, be1_ref, w2_ref, b2_ref,
              g2_ref, be2_ref, w3_ref, b3_ref, o_ref, t_ref):
    for i in range(NB):
        r = pl.ds(i * RB, RB)
        rt = pl.ds(N + i * RB, RB)
        u = jnp.concatenate([x_ref[r, :], x_ref[rt, :]], axis=1)
        t_ref[r, :] = _dot_mx(u, w1_ref[...]) + b1_ref[...]
    mu, var = _bn_stats(t_ref)
    h = _leaky(_bn_apply(t_ref[...], mu, var, g1_ref[...], be1_ref[...]))
    for i in range(NB):
        r = pl.ds(i * RB, RB)
        t_ref[r, :] = _dot_mx(h[i * RB:(i + 1) * RB, :], w2_ref[...]) + b2_ref[...]
    mu2, var2 = _bn_stats(t_ref)
    h2 = _leaky(_bn_apply(t_ref[...], mu2, var2, g2_ref[...], be2_ref[...]))
    h2b = h2.astype(jnp.bfloat16).astype(jnp.float32)
    w3b = w3_ref[...].astype(jnp.bfloat16).astype(jnp.float32)
    o_ref[...] = jnp.sum(h2b * w3b, axis=1, keepdims=True) + b3_ref[0, 0]


@jax.jit
def _tc_cls(x_flat, w1, b1, g1, be1, w2, b2, g2, be2, w3, b3):
    return pl.pallas_call(
        _cls_body,
        out_shape=jax.ShapeDtypeStruct((N, 1), jnp.float32),
        scratch_shapes=[pltpu.VMEM((N, D), jnp.float32)],
    )(x_flat, w1, b1.reshape(1, D), g1.reshape(1, D), be1.reshape(1, D),
      w2, b2.reshape(1, D), g2.reshape(1, D), be2.reshape(1, D),
      w3.reshape(1, D), b3.reshape(1, 1))


def kernel(x, edge_index, params):
    src = edge_index[0].astype(jnp.int32)
    dst = edge_index[1].astype(jnp.int32)

    bucket = dst // NPB
    onehot = (bucket[:, None] == jnp.arange(NS, dtype=jnp.int32)[None, :]).astype(jnp.int32)
    rank = jnp.sum(jnp.cumsum(onehot, axis=0) * onehot, axis=1) - 1
    pos = jnp.where(rank < P, bucket * P + rank, NS * P)
    fill_src = (jnp.arange(NS * P, dtype=jnp.int32) * 7919) % N
    src_pad = fill_src.at[pos].set(src, unique_indices=True, mode="drop")
    dst_pad = jnp.full((NS * P,), TRASH, jnp.int32).at[pos].set(
        dst, unique_indices=True, mode="drop")
    src2 = jnp.concatenate([src_pad, src_pad + N])

    x_flat = jnp.concatenate([x[:, :H], x[:, H:]], axis=0)
    zeros = jnp.zeros((N, H), jnp.float32)

    for i in range(3):
        agg_flat = _sc_scatter(x_flat, src2, dst_pad, zeros)
        h1 = _tc_mm1(x_flat, agg_flat, params[f"conv{i}_W1"],
                     params[f"conv{i}_b1"], params[f"conv{i}_eps"])
        x_flat = _tc_bn_mm2(h1, params[f"conv{i}_g1"], params[f"conv{i}_be1"],
                            params[f"conv{i}_W2"], params[f"conv{i}_b2"],
                            params.get(f"bn{i}_g", params[f"conv{i}_g1"]),
                            params.get(f"bn{i}_b", params[f"conv{i}_be1"]),
                            do_bn2=(i != 2))

    out = _tc_cls(x_flat, params["cls_W1"], params["cls_b1"],
                  params["cls_g1"], params["cls_be1"],
                  params["cls_W2"], params["cls_b2"],
                  params["cls_g2"], params["cls_be2"],
                  params["cls_W3"], params["cls_b3"])
    return out.reshape(-1)


# speedup vs baseline: 1.9792x; 1.5107x over previous
import functools

import jax
import jax.numpy as jnp
from jax import lax
from jax.experimental import pallas as pl
from jax.experimental.pallas import tpu as pltpu
from jax.experimental.pallas import tpu_sc as plsc

N = 10000
E = 160000
D = 256
H = 128
NS = 16
NPB = 625
P = 10800
K = 80
NCHUNK = P // K
TRASH = N
AROWS = N + 16
RPT = 624


def _sc_scatter_body(x_hbm, packed_hbm, zeros_hbm, out_hbm,
                     pk_v, src_v, dst_v, rows_v, agg_sh, sem):
    c = lax.axis_index("c")
    s = lax.axis_index("s")

    pltpu.sync_copy(zeros_hbm.at[pl.ds(s * RPT, RPT)],
                    agg_sh.at[pl.ds(s * RPT, RPT)])

    @pl.when(s < 2)
    def _():
        pltpu.sync_copy(zeros_hbm.at[pl.ds(NS * RPT + s * 8, 8)],
                        agg_sh.at[pl.ds(NS * RPT + s * 8, 8)])

    plsc.subcore_barrier()

    ebase = s * P

    cN = c * N

    @pl.loop(0, NCHUNK)
    def _(j):
        base = ebase + j * K
        pltpu.sync_copy(packed_hbm.at[pl.ds(base, K)], pk_v)
        for i in range(K // 16):
            sl = pl.ds(i * 16, 16)
            p = pk_v[sl]
            src_v[sl] = (p >> 14) + cN
            dst_v[sl] = p & 16383
        pltpu.async_copy(x_hbm.at[src_v], rows_v, sem).wait()
        pltpu.sync_copy(rows_v, agg_sh.at[dst_v], add=True)

    plsc.subcore_barrier()
    pltpu.sync_copy(agg_sh.at[pl.ds(s * RPT, RPT)],
                    out_hbm.at[pl.ds(c * N + s * RPT, RPT)])

    @pl.when(s < 2)
    def _():
        pltpu.sync_copy(agg_sh.at[pl.ds(NS * RPT + s * 8, 8)],
                        out_hbm.at[pl.ds(c * N + NS * RPT + s * 8, 8)])


@jax.jit
def _sc_scatter(x_flat, packed, zeros):
    mesh = plsc.VectorSubcoreMesh(core_axis_name="c", subcore_axis_name="s",
                                  num_cores=2, num_subcores=NS)
    fn = pl.kernel(
        _sc_scatter_body,
        out_type=jax.ShapeDtypeStruct((2 * N, H), jnp.float32),
        mesh=mesh,
        scratch_types=[
            pltpu.VMEM((K,), jnp.int32),
            pltpu.VMEM((K,), jnp.int32),
            pltpu.VMEM((K,), jnp.int32),
            pltpu.VMEM((K, H), jnp.float32),
            pltpu.VMEM_SHARED((AROWS, H), jnp.float32),
            pltpu.SemaphoreType.DMA,
        ],
    )
    return fn(x_flat, packed, zeros)


RB = 2000
NB = N // RB


def _dot_mx(a, b):
    return jnp.dot(a, b, preferred_element_type=jnp.float32)


def _leaky(h):
    return jnp.where(h >= 0.0, h, 0.01 * h)


def _col_sum(h_ref, f):
    def half(lo):
        def body(i, acc):
            return acc + f(h_ref[pl.ds(lo + i * 8, 8), :])
        acc = lax.fori_loop(0, (N // 2) // 8, body,
                            jnp.zeros((8, D), jnp.float32), unroll=25)
        t = acc[0:4, :] + acc[4:8, :]
        t = t[0:2, :] + t[2:4, :]
        return t[0:1, :] + t[1:2, :]
    return half(0) + half(N // 2)


def _bn_stats(h_ref):
    mu = _col_sum(h_ref, lambda t: t) * (1.0 / N)
    var = _col_sum(h_ref, lambda t: (t - mu) * (t - mu)) * (1.0 / N)
    return mu, var


def _bn_apply(h, mu, var, g, b):
    return (h - mu) / jnp.sqrt(var + 1e-5) * g + b


def _mm1_body(x_ref, a_ref, w_ref, b_ref, eps_ref, o_ref):
    e = 1.0 + eps_ref[0, 0]
    for i in range(NB):
        r = pl.ds(i * RB, RB)
        rt = pl.ds(N + i * RB, RB)
        u = jnp.concatenate([e * x_ref[r, :] + a_ref[r, :],
                             e * x_ref[rt, :] + a_ref[rt, :]], axis=1)
        o_ref[r, :] = _dot_mx(u, w_ref[...]) + b_ref[...]


@jax.jit
def _tc_mm1(x_flat, agg_flat, w1, b1, eps):
    return pl.pallas_call(
        _mm1_body,
        out_shape=jax.ShapeDtypeStruct((N, D), jnp.float32),
    )(x_flat, agg_flat, w1, b1.reshape(1, D), eps.reshape(1, 1))


def _bn_mm2_body(h_ref, g1_ref, be1_ref, w2_ref, b2_ref, g2_ref, be2_ref,
                 o_ref, y_ref, *, do_bn2):
    mu, var = _bn_stats(h_ref)
    h = _leaky(_bn_apply(h_ref[...], mu, var, g1_ref[...], be1_ref[...]))
    for i in range(NB):
        r = pl.ds(i * RB, RB)
        y_ref[r, :] = _dot_mx(h[i * RB:(i + 1) * RB, :], w2_ref[...]) + b2_ref[...]
    y = y_ref[...]
    if do_bn2:
        mu2, var2 = _bn_stats(y_ref)
        y = _bn_apply(y, mu2, var2, g2_ref[...], be2_ref[...])
    y = _leaky(y)
    o_ref[:N, :] = y[:, :H]
    o_ref[N:, :] = y[:, H:]


@functools.partial(jax.jit, static_argnames=("do_bn2",))
def _tc_bn_mm2(h1, g1, be1, w2, b2, g2, be2, do_bn2):
    return pl.pallas_call(
        functools.partial(_bn_mm2_body, do_bn2=do_bn2),
        out_shape=jax.ShapeDtypeStruct((2 * N, H), jnp.float32),
        scratch_shapes=[pltpu.VMEM((N, D), jnp.float32)],
    )(h1, g1.reshape(1, D), be1.reshape(1, D), w2, b2.reshape(1, D),
      g2.reshape(1, D), be2.reshape(1, D))


def _cls_body(x_ref, w1_ref, b1_ref, g1_ref, be1_ref, w2_ref, b2_ref,
              g2_ref, be2_ref, w3_ref, b3_ref, o_ref, t_ref):
    for i in range(NB):
        r = pl.ds(i * RB, RB)
        rt = pl.ds(N + i * RB, RB)
        u = jnp.concatenate([x_ref[r, :], x_ref[rt, :]], axis=1)
        t_ref[r, :] = _dot_mx(u, w1_ref[...]) + b1_ref[...]
    mu, var = _bn_stats(t_ref)
    h = _leaky(_bn_apply(t_ref[...], mu, var, g1_ref[...], be1_ref[...]))
    for i in range(NB):
        r = pl.ds(i * RB, RB)
        t_ref[r, :] = _dot_mx(h[i * RB:(i + 1) * RB, :], w2_ref[...]) + b2_ref[...]
    mu2, var2 = _bn_stats(t_ref)
    h2 = _leaky(_bn_apply(t_ref[...], mu2, var2, g2_ref[...], be2_ref[...]))
    h2b = h2.astype(jnp.bfloat16).astype(jnp.float32)
    w3b = w3_ref[...].astype(jnp.bfloat16).astype(jnp.float32)
    o_ref[...] = jnp.sum(h2b * w3b, axis=1, keepdims=True) + b3_ref[0, 0]


@jax.jit
def _tc_cls(x_flat, w1, b1, g1, be1, w2, b2, g2, be2, w3, b3):
    return pl.pallas_call(
        _cls_body,
        out_shape=jax.ShapeDtypeStruct((N, 1), jnp.float32),
        scratch_shapes=[pltpu.VMEM((N, D), jnp.float32)],
    )(x_flat, w1, b1.reshape(1, D), g1.reshape(1, D), be1.reshape(1, D),
      w2, b2.reshape(1, D), g2.reshape(1, D), be2.reshape(1, D),
      w3.reshape(1, D), b3.reshape(1, 1))


def kernel(x, edge_index, params):
    src = edge_index[0].astype(jnp.int32)
    dst = edge_index[1].astype(jnp.int32)

    bucket = dst // NPB
    onehot = (bucket[:, None] == jnp.arange(NS, dtype=jnp.int32)[None, :]).astype(jnp.int32)
    rank = jnp.sum(jnp.cumsum(onehot, axis=0) * onehot, axis=1) - 1
    pos = jnp.where(rank < P, bucket * P + rank, NS * P)
    fill = ((jnp.arange(NS * P, dtype=jnp.int32) * 7919) % N) * 16384 + TRASH
    packed = fill.at[pos].set(src * 16384 + dst, unique_indices=True,
                              mode="drop")

    x_flat = jnp.concatenate([x[:, :H], x[:, H:]], axis=0)
    zeros = jnp.zeros((N, H), jnp.float32)

    for i in range(3):
        agg_flat = _sc_scatter(x_flat, packed, zeros)
        h1 = _tc_mm1(x_flat, agg_flat, params[f"conv{i}_W1"],
                     params[f"conv{i}_b1"], params[f"conv{i}_eps"])
        x_flat = _tc_bn_mm2(h1, params[f"conv{i}_g1"], params[f"conv{i}_be1"],
                            params[f"conv{i}_W2"], params[f"conv{i}_b2"],
                            params.get(f"bn{i}_g", params[f"conv{i}_g1"]),
                            params.get(f"bn{i}_b", params[f"conv{i}_be1"]),
                            do_bn2=(i != 2))

    out = _tc_cls(x_flat, params["cls_W1"], params["cls_b1"],
                  params["cls_g1"], params["cls_be1"],
                  params["cls_W2"], params["cls_b2"],
                  params["cls_g2"], params["cls_be2"],
                  params["cls_W3"], params["cls_b3"])
    return out.reshape(-1)
